# baseline (speedup 1.0000x reference)
import jax
import jax.numpy as jnp
from jax.experimental import pallas as pl
from jax.experimental.pallas import tpu as pltpu

_B = 4096
_D = 1024
_K = 50
_P = 10
_SIGMA = 0.5
_ALPHA = 0.1

_RB = 256
_C = 256
_NCH = _B // _C
_RBP = 512
_NP = _B // _RBP
_NB = _B // _RB
_NK = _NB // 2


def _softplus(x):
    return jnp.maximum(x, 0.0) + jnp.log(1.0 + jnp.exp(-jnp.abs(x)))


def _dot_t(a, b):
    return jax.lax.dot_general(a, b, (((1,), (1,)), ((), ())),
                               preferred_element_type=jnp.float32)


def _count_block(zh_ref, zl_ref, src_ref, dst_ref, nxt_blk):
    zrh = zh_ref[pl.ds(nxt_blk * _RB, _RB), :]
    zrl = zl_ref[pl.ds(nxt_blk * _RB, _RB), :]

    def it(t, carry):
        lo, hi = carry
        zch = zh_ref[pl.ds(t * _C, _C), :]
        zcl = zl_ref[pl.ds(t * _C, _C), :]
        g = (_dot_t(zrh, zch) + _dot_t(zrh, zcl) + _dot_t(zrl, zch))
        dst_ref[:, pl.ds(t * _C, _C)] = g
        mid = 0.5 * (lo + hi)
        thr = 1.0 - 0.5 * mid
        cnt = jnp.sum((src_ref[...] >= thr).astype(jnp.float32),
                      axis=1, keepdims=True)
        ge = cnt >= float(_K + 1)
        return jnp.where(ge, lo, mid), jnp.where(ge, mid, hi)

    lo0 = jnp.zeros((_RB, 1), jnp.float32)
    hi0 = jnp.full((_RB, 1), 4.5, jnp.float32)
    _, hi = jax.lax.fori_loop(0, _NCH, it, (lo0, hi0))
    return jnp.sqrt(hi[:, 0])


def _mega_body(emb_ref, w_ref, noise_ref, b_ref, out_ref,
               zh_ref, zl_ref, zw_ref, buf_a, buf_b, knn_ref):
    s = pl.program_id(0)

    @pl.when(s < _NP)
    def _prep():
        x = emb_ref[...]
        ss = jnp.sum(x * x, axis=1, keepdims=True)
        norm = jnp.maximum(jnp.sqrt(ss), 1e-12)
        z = x / norm
        zh = z.astype(jnp.bfloat16)
        zl = (z - zh.astype(jnp.float32)).astype(jnp.bfloat16)
        zh_ref[pl.ds(s * _RBP, _RBP), :] = zh
        zl_ref[pl.ds(s * _RBP, _RBP), :] = zl
        zw_ref[s] = jax.lax.dot_general(
            z, w_ref[...], (((1,), (0,)), ((), ())),
            preferred_element_type=jnp.float32)[:, 0]

    @pl.when(s == _NP - 1)
    def _prime():
        zrh = zh_ref[pl.ds(0, _RB), :]
        zrl = zl_ref[pl.ds(0, _RB), :]
        buf_a[...] = (_dot_t(zrh, zh_ref[...]) + _dot_t(zrh, zl_ref[...])
                      + _dot_t(zrl, zh_ref[...]))

    @pl.when(jnp.logical_and(s >= _NP, s < _NP + _NK))
    def _knn():
        k2 = s - _NP
        blk = 2 * k2
        knn_ref[blk] = _count_block(zh_ref, zl_ref, buf_a, buf_b, blk + 1)
        knn_ref[blk + 1] = _count_block(
            zh_ref, zl_ref, buf_b, buf_a, jnp.minimum(blk + 2, _NB - 1))

    @pl.when(s == _NP + _NK)
    def _final():
        bval = b_ref[0]
        zw = zw_ref[...]
        gw = jax.lax.dot_general(
            noise_ref[...], w_ref[...], (((1,), (0,)), ((), ())),
            preferred_element_type=jnp.float32)
        id_loss = jnp.sum(_softplus(-(zw + bval))) / float(_B)
        idx_zw = (jax.lax.broadcasted_iota(jnp.int32, (_NP, _RBP), 0) * _RBP
                  + jax.lax.broadcasted_iota(jnp.int32, (_NP, _RBP), 1))
        idx_kn = (jax.lax.broadcasted_iota(jnp.int32, (_NB, _RB), 0) * _RB
                  + jax.lax.broadcasted_iota(jnp.int32, (_NB, _RB), 1))
        v = knn_ref[...]
        ood_sum = jnp.float32(0.0)
        for p in range(_P):
            m = jnp.max(v)
            gidx = jnp.min(jnp.where(v == m, idx_kn, _B))
            zsel = jnp.sum(jnp.where(idx_zw == gidx, zw, 0.0))
            ood_sum = ood_sum + _softplus(zsel + bval + _SIGMA * gw[p, 0])
            v = jnp.where(idx_kn == gidx, -1.0, v)
        out = _ALPHA * (id_loss + ood_sum / float(_P))
        out_ref[...] = jnp.full((1, 1), out, jnp.float32)


def kernel(embeddings, labels, W, b):
    del labels
    emb = embeddings.astype(jnp.float32)
    w = W.astype(jnp.float32)
    noise = jax.random.normal(jax.random.key(1234), (_P, 1, _D),
                              dtype=jnp.float32).reshape(_P, _D)
    out = pl.pallas_call(
        _mega_body,
        grid=(_NP + _NK + 1,),
        in_specs=[
            pl.BlockSpec((_RBP, _D), lambda s: (jnp.minimum(s, _NP - 1), 0)),
            pl.BlockSpec((_D, 1), lambda s: (0, 0)),
            pl.BlockSpec((_P, _D), lambda s: (0, 0)),
            pl.BlockSpec(memory_space=pltpu.SMEM),
        ],
        out_specs=pl.BlockSpec((1, 1), lambda s: (0, 0)),
        out_shape=jax.ShapeDtypeStruct((1, 1), jnp.float32),
        scratch_shapes=[
            pltpu.VMEM((_B, _D), jnp.bfloat16),
            pltpu.VMEM((_B, _D), jnp.bfloat16),
            pltpu.VMEM((_NP, _RBP), jnp.float32),
            pltpu.VMEM((_RB, _B), jnp.float32),
            pltpu.VMEM((_RB, _B), jnp.float32),
            pltpu.VMEM((_NB, _RB), jnp.float32),
        ],
    )(emb, w, noise, b.astype(jnp.float32))
    return out.reshape(())


# speedup vs baseline: 1.3665x; 1.3665x over previous
import jax
import jax.numpy as jnp
from jax.experimental import pallas as pl
from jax.experimental.pallas import tpu as pltpu

_B = 4096
_D = 1024
_K = 50
_P = 10
_SIGMA = 0.5
_ALPHA = 0.1

_RB = 256
_C = 256
_NCH = _B // _C
_RBP = 512
_NP = _B // _RBP
_NB = _B // _RB
_NK = _NB // 2


def _softplus(x):
    return jnp.maximum(x, 0.0) + jnp.log(1.0 + jnp.exp(-jnp.abs(x)))


def _dot_t(a, b):
    return jax.lax.dot_general(a, b, (((1,), (1,)), ((), ())),
                               preferred_element_type=jnp.float32)


_NBIS = 13


def _count_block(z_ref, src_ref, dst_ref, nxt_blk):
    zr = z_ref[pl.ds(nxt_blk * _RB, _RB), :]

    def mm(t):
        zc = z_ref[pl.ds(t * _C, _C), :]
        dst_ref[:, pl.ds(t * _C, _C)] = _dot_t(zr, zc)

    def it(t, carry):
        lo, hi = carry
        mm(t)
        mid = 0.5 * (lo + hi)
        thr = 1.0 - 0.5 * mid
        cnt = jnp.sum((src_ref[...] >= thr).astype(jnp.float32),
                      axis=1, keepdims=True)
        ge = cnt >= float(_K + 1)
        return jnp.where(ge, lo, mid), jnp.where(ge, mid, hi)

    def it_mm(t, carry):
        mm(t)
        return carry

    lo0 = jnp.zeros((_RB, 1), jnp.float32)
    hi0 = jnp.full((_RB, 1), 4.5, jnp.float32)
    carry = jax.lax.fori_loop(0, _NBIS, it, (lo0, hi0))
    _, hi = jax.lax.fori_loop(_NBIS, _NCH, it_mm, carry)
    return jnp.sqrt(hi[:, 0])


def _mega_body(emb_ref, w_ref, noise_ref, b_ref, out_ref,
               z_ref, zw_ref, buf_a, buf_b, knn_ref):
    s = pl.program_id(0)

    @pl.when(s < _NP)
    def _prep():
        x = emb_ref[...]
        ss = jnp.sum(x * x, axis=1, keepdims=True)
        norm = jnp.maximum(jnp.sqrt(ss), 1e-12)
        z = x / norm
        z_ref[pl.ds(s * _RBP, _RBP), :] = z
        zw_ref[s] = jax.lax.dot_general(
            z, w_ref[...], (((1,), (0,)), ((), ())),
            preferred_element_type=jnp.float32)[:, 0]

    @pl.when(s == _NP - 1)
    def _prime():
        buf_a[...] = _dot_t(z_ref[pl.ds(0, _RB), :], z_ref[...])

    @pl.when(jnp.logical_and(s >= _NP, s < _NP + _NK))
    def _knn():
        k2 = s - _NP
        blk = 2 * k2
        knn_ref[blk] = _count_block(z_ref, buf_a, buf_b, blk + 1)
        knn_ref[blk + 1] = _count_block(
            z_ref, buf_b, buf_a, jnp.minimum(blk + 2, _NB - 1))

    @pl.when(s == _NP + _NK)
    def _final():
        bval = b_ref[0]
        zw = zw_ref[...]
        gw = jax.lax.dot_general(
            noise_ref[...], w_ref[...], (((1,), (0,)), ((), ())),
            preferred_element_type=jnp.float32)
        id_loss = jnp.sum(_softplus(-(zw + bval))) / float(_B)
        idx_zw = (jax.lax.broadcasted_iota(jnp.int32, (_NP, _RBP), 0) * _RBP
                  + jax.lax.broadcasted_iota(jnp.int32, (_NP, _RBP), 1))
        idx_kn = (jax.lax.broadcasted_iota(jnp.int32, (_NB, _RB), 0) * _RB
                  + jax.lax.broadcasted_iota(jnp.int32, (_NB, _RB), 1))
        v = knn_ref[...]
        ood_sum = jnp.float32(0.0)
        for p in range(_P):
            m = jnp.max(v)
            gidx = jnp.min(jnp.where(v == m, idx_kn, _B))
            zsel = jnp.sum(jnp.where(idx_zw == gidx, zw, 0.0))
            ood_sum = ood_sum + _softplus(zsel + bval + _SIGMA * gw[p, 0])
            v = jnp.where(idx_kn == gidx, -1.0, v)
        out = _ALPHA * (id_loss + ood_sum / float(_P))
        out_ref[...] = jnp.full((1, 1), out, jnp.float32)


def kernel(embeddings, labels, W, b):
    del labels
    emb = embeddings.astype(jnp.float32)
    w = W.astype(jnp.float32)
    noise = jax.random.normal(jax.random.key(1234), (_P, 1, _D),
                              dtype=jnp.float32).reshape(_P, _D)
    out = pl.pallas_call(
        _mega_body,
        grid=(_NP + _NK + 1,),
        in_specs=[
            pl.BlockSpec((_RBP, _D), lambda s: (jnp.minimum(s, _NP - 1), 0)),
            pl.BlockSpec((_D, 1), lambda s: (0, 0)),
            pl.BlockSpec((_P, _D), lambda s: (0, 0)),
            pl.BlockSpec(memory_space=pltpu.SMEM),
        ],
        out_specs=pl.BlockSpec((1, 1), lambda s: (0, 0)),
        out_shape=jax.ShapeDtypeStruct((1, 1), jnp.float32),
        scratch_shapes=[
            pltpu.VMEM((_B, _D), jnp.float32),
            pltpu.VMEM((_NP, _RBP), jnp.float32),
            pltpu.VMEM((_RB, _B), jnp.float32),
            pltpu.VMEM((_RB, _B), jnp.float32),
            pltpu.VMEM((_NB, _RB), jnp.float32),
        ],
    )(emb, w, noise, b.astype(jnp.float32))
    return out.reshape(())
